# baseline (speedup 1.0000x reference)
import functools

import jax
import jax.numpy as jnp
from jax import lax
from jax.experimental import pallas as pl
from jax.experimental.pallas import tpu as pltpu
from jax.experimental.pallas import tpu_sc as plsc

H = 32
V = 1537
VP = 1544
D = 5
F = 3
K = D * F
B, N = 16, 64
P = N * N
C = 128
HH = H // 2
TW = D * VP


def _fold_body(we_ref, w_ref, out_ref):
    out_ref[0] = jnp.dot(we_ref[...], w_ref[0], preferred_element_type=jnp.float32)


def _fold_tables(we_pad, w):
    return pl.pallas_call(
        _fold_body,
        grid=(D,),
        in_specs=[
            pl.BlockSpec((VP, H), lambda d: (0, 0)),
            pl.BlockSpec((1, H, H), lambda d: (d, 0, 0)),
        ],
        out_specs=pl.BlockSpec((1, VP, H), lambda d: (d, 0, 0)),
        out_shape=jax.ShapeDtypeStruct((D, VP, H), jnp.float32),
    )(we_pad, w)


_mesh = plsc.VectorSubcoreMesh(core_axis_name="c", subcore_axis_name="s")


@functools.partial(
    pl.kernel,
    out_type=jax.ShapeDtypeStruct((B, H, P), jnp.float32),
    mesh=_mesh,
    scratch_types=[
        pltpu.VMEM((HH * TW,), jnp.float32),
        pltpu.VMEM((C * K,), jnp.int32),
        pltpu.VMEM((C,), jnp.int32),
        pltpu.VMEM((HH, C), jnp.float32),
    ],
    compiler_params=pltpu.CompilerParams(needs_layout_passes=False),
)
def _sc_kernel(tab_hbm, idx_hbm, sp_hbm, out_hbm, tab_v, idx_v, sp_v, out_v):
    cid = lax.axis_index("c")
    sid = lax.axis_index("s")
    wid = sid * 2 + cid
    b = wid % B
    half = wid // B
    pltpu.sync_copy(tab_hbm.at[pl.ds(half * (HH * TW), HH * TW)], tab_v)
    iota = lax.iota(jnp.int32, 16)
    iota_k = iota * K

    def chunk_body(chunk, _):
        base = chunk * C
        pltpu.sync_copy(idx_hbm.at[b, pl.ds(base * K, C * K)], idx_v)
        pltpu.sync_copy(sp_hbm.at[b, pl.ds(base, C)], sp_v)

        @plsc.parallel_loop(0, C // 16, unroll=4)
        def group_body(g):
            pos = g * 16 + iota
            vks = [
                plsc.load_gather(idx_v, [g * (16 * K) + k + iota_k])
                + (k // 3) * VP
                for k in range(K)
            ]
            s = plsc.load_gather(sp_v, [pos])
            s = jnp.where(s == 0, 1, s)
            s = jnp.where(s > 1, s - 1, s)
            s = jnp.minimum(jnp.maximum(s, 0), D)
            scale = 1.0 / (3.0 * s.astype(jnp.float32))
            for h in range(HH):
                vals = [plsc.load_gather(tab_v, [vks[k] + h * TW]) for k in range(K)]
                while len(vals) > 1:
                    vals = [vals[i] + vals[i + 1] for i in range(0, len(vals) - 1, 2)] + (
                        [vals[-1]] if len(vals) % 2 else [])
                out_v[h, pl.ds(g * 16, 16)] = vals[0] * scale

        pltpu.sync_copy(out_v, out_hbm.at[b, pl.ds(half * HH, HH), pl.ds(base, C)])
        return 0

    lax.fori_loop(0, P // C, chunk_body, 0)


def kernel(attn_bias, spatial_pos, x, edge_input, attn_edge_type,
           edge_encoder_weight, edge_dis_encoder_weight):
    we_pad = jnp.pad(edge_encoder_weight, ((0, VP - V), (0, 0)))
    w = edge_dis_encoder_weight.reshape(-1, H, H)[:D]
    tc = _fold_tables(we_pad, w)
    tab = tc.transpose(2, 0, 1).reshape(H * TW)
    idx = edge_input.reshape(B, P * K).astype(jnp.int32)
    sp = spatial_pos.reshape(B, P).astype(jnp.int32)
    out = _sc_kernel(tab, idx, sp)
    return out.reshape(B, H, N, N)


# speedup vs baseline: 1.0027x; 1.0027x over previous
import functools

import jax
import jax.numpy as jnp
from jax import lax
from jax.experimental import pallas as pl
from jax.experimental.pallas import tpu as pltpu
from jax.experimental.pallas import tpu_sc as plsc

H = 32
V = 1537
VP = 1544
D = 5
F = 3
K = D * F
B, N = 16, 64
P = N * N
C = 128
HH = H // 2
TW = D * VP


def _fold_body(we_ref, w_ref, out_ref):
    out_ref[0] = jnp.dot(we_ref[...], w_ref[0], preferred_element_type=jnp.float32)


def _fold_tables(we_pad, w):
    return pl.pallas_call(
        _fold_body,
        grid=(D,),
        in_specs=[
            pl.BlockSpec((VP, H), lambda d: (0, 0)),
            pl.BlockSpec((1, H, H), lambda d: (d, 0, 0)),
        ],
        out_specs=pl.BlockSpec((1, VP, H), lambda d: (d, 0, 0)),
        out_shape=jax.ShapeDtypeStruct((D, VP, H), jnp.float32),
    )(we_pad, w)


_mesh = plsc.VectorSubcoreMesh(core_axis_name="c", subcore_axis_name="s")


@functools.partial(
    pl.kernel,
    out_type=jax.ShapeDtypeStruct((B, H, P), jnp.float32),
    mesh=_mesh,
    scratch_types=[
        pltpu.VMEM((HH * TW,), jnp.float32),
        pltpu.VMEM((C * K,), jnp.int32),
        pltpu.VMEM((C,), jnp.int32),
        pltpu.VMEM((HH, C), jnp.float32),
    ],
    compiler_params=pltpu.CompilerParams(needs_layout_passes=False),
)
def _sc_kernel(tab_hbm, idx_hbm, sp_hbm, out_hbm, tab_v, idx_v, sp_v, out_v):
    cid = lax.axis_index("c")
    sid = lax.axis_index("s")
    wid = sid * 2 + cid
    b = wid % B
    half = wid // B
    pltpu.sync_copy(tab_hbm.at[pl.ds(half * (HH * TW), HH * TW)], tab_v)
    iota = lax.iota(jnp.int32, 16)
    iota_k = iota * K

    def chunk_body(chunk, _):
        base = chunk * C
        pltpu.sync_copy(idx_hbm.at[b, pl.ds(base * K, C * K)], idx_v)
        pltpu.sync_copy(sp_hbm.at[b, pl.ds(base, C)], sp_v)

        @plsc.parallel_loop(0, C // 16, unroll=4)
        def group_body(g):
            pos = g * 16 + iota
            vks = [
                plsc.load_gather(idx_v, [g * (16 * K) + k + iota_k])
                + (k // 3) * VP
                for k in range(K)
            ]
            s = plsc.load_gather(sp_v, [pos])
            s = jnp.where(s == 0, 1, s)
            s = jnp.where(s > 1, s - 1, s)
            s = jnp.minimum(jnp.maximum(s, 0), D)
            scale = 1.0 / (3.0 * s.astype(jnp.float32))
            for h in range(HH):
                hrow = tab_v.at[pl.ds(h * TW, TW)]
                vals = [plsc.load_gather(hrow, [vks[k]]) for k in range(K)]
                while len(vals) > 1:
                    vals = [vals[i] + vals[i + 1] for i in range(0, len(vals) - 1, 2)] + (
                        [vals[-1]] if len(vals) % 2 else [])
                out_v[h, pl.ds(g * 16, 16)] = vals[0] * scale

        pltpu.sync_copy(out_v, out_hbm.at[b, pl.ds(half * HH, HH), pl.ds(base, C)])
        return 0

    lax.fori_loop(0, P // C, chunk_body, 0)


def kernel(attn_bias, spatial_pos, x, edge_input, attn_edge_type,
           edge_encoder_weight, edge_dis_encoder_weight):
    we_pad = jnp.pad(edge_encoder_weight, ((0, VP - V), (0, 0)))
    w = edge_dis_encoder_weight.reshape(-1, H, H)[:D]
    tc = _fold_tables(we_pad, w)
    tab = tc.transpose(2, 0, 1).reshape(H * TW)
    idx = edge_input.reshape(B, P * K).astype(jnp.int32)
    sp = spatial_pos.reshape(B, P).astype(jnp.int32)
    out = _sc_kernel(tab, idx, sp)
    return out.reshape(B, H, N, N)


# speedup vs baseline: 1.2203x; 1.2170x over previous
import functools

import jax
import jax.numpy as jnp
from jax import lax
from jax.experimental import pallas as pl
from jax.experimental.pallas import tpu as pltpu
from jax.experimental.pallas import tpu_sc as plsc

H = 32
V = 1537
VP = 1544
D = 5
F = 3
K = D * F
B, N = 16, 64
P = N * N
C = 128
HH = H // 2
TW = D * VP
PW = P // 2


def _fold_body(we_ref, w_ref, out_ref):
    out_ref[0] = jnp.dot(we_ref[...], w_ref[0], preferred_element_type=jnp.float32)


def _fold_tables(we_pad, w):
    return pl.pallas_call(
        _fold_body,
        grid=(D,),
        in_specs=[
            pl.BlockSpec((VP, H), lambda d: (0, 0)),
            pl.BlockSpec((1, H, H), lambda d: (d, 0, 0)),
        ],
        out_specs=pl.BlockSpec((1, VP, H), lambda d: (d, 0, 0)),
        out_shape=jax.ShapeDtypeStruct((D, VP, H), jnp.float32),
    )(we_pad, w)


_mesh = plsc.VectorSubcoreMesh(core_axis_name="c", subcore_axis_name="s")


@functools.partial(
    pl.kernel,
    out_type=jax.ShapeDtypeStruct((B, H, P), jnp.float32),
    mesh=_mesh,
    scratch_types=[
        pltpu.VMEM((TW * HH,), jnp.int32),
        pltpu.VMEM((C * K,), jnp.int32),
        pltpu.VMEM((C,), jnp.int32),
        pltpu.VMEM((H, C), jnp.float32),
    ],
    compiler_params=pltpu.CompilerParams(needs_layout_passes=False),
)
def _sc_kernel(tab_hbm, idx_hbm, sp_hbm, out_hbm, tab_v, idx_v, sp_v, out_v):
    cid = lax.axis_index("c")
    sid = lax.axis_index("s")
    wid = sid * 2 + cid
    b = wid % B
    ph = wid // B
    pltpu.sync_copy(tab_hbm, tab_v)
    iota = lax.iota(jnp.int32, 16)
    iota_k = iota * K

    def chunk_body(chunk, _):
        base = ph * PW + chunk * C
        pltpu.sync_copy(idx_hbm.at[b, pl.ds(base * K, C * K)], idx_v)
        pltpu.sync_copy(sp_hbm.at[b, pl.ds(base, C)], sp_v)

        @plsc.parallel_loop(0, C // 16, unroll=2)
        def group_body(g):
            pos = g * 16 + iota
            vks = [
                plsc.load_gather(idx_v, [g * (16 * K) + k + iota_k])
                + (k // 3) * VP
                for k in range(K)
            ]
            s = plsc.load_gather(sp_v, [pos])
            s = jnp.where(s == 0, 1, s)
            s = jnp.where(s > 1, s - 1, s)
            s = jnp.minimum(jnp.maximum(s, 0), D)
            scale = 1.0 / (3.0 * s.astype(jnp.float32))
            for hp in range(HH):
                hrow = tab_v.at[pl.ds(hp * TW, TW)]
                words = [plsc.bitcast(plsc.load_gather(hrow, [vks[k]]),
                                      jnp.bfloat16)
                         for k in range(K)]
                paired = [words[i] + words[i + 1] for i in range(0, K - 1, 2)]
                paired.append(words[K - 1])
                los, his = [], []
                for wv in paired:
                    lo, hi = plsc.unpack(wv, format=plsc.PackFormat.INTERLEAVED)
                    los.append(lo)
                    his.append(hi)
                for vals in (los, his):
                    while len(vals) > 1:
                        vals[:] = [vals[i] + vals[i + 1]
                                   for i in range(0, len(vals) - 1, 2)] + (
                            [vals[-1]] if len(vals) % 2 else [])
                out_v[hp, pl.ds(g * 16, 16)] = los[0] * scale
                out_v[hp + HH, pl.ds(g * 16, 16)] = his[0] * scale

        pltpu.sync_copy(out_v, out_hbm.at[b, pl.ds(0, H), pl.ds(base, C)])
        return 0

    lax.fori_loop(0, PW // C, chunk_body, 0)


def kernel(attn_bias, spatial_pos, x, edge_input, attn_edge_type,
           edge_encoder_weight, edge_dis_encoder_weight):
    we_pad = jnp.pad(edge_encoder_weight, ((0, VP - V), (0, 0)))
    w = edge_dis_encoder_weight.reshape(-1, H, H)[:D]
    tc = _fold_tables(we_pad, w).reshape(TW, H)
    pairs = jnp.stack([tc[:, :HH], tc[:, HH:]], axis=-1).astype(jnp.bfloat16)
    tab = lax.bitcast_convert_type(
        pairs.transpose(1, 0, 2), jnp.int32).reshape(HH * TW)
    idx = edge_input.reshape(B, P * K).astype(jnp.int32)
    sp = spatial_pos.reshape(B, P).astype(jnp.int32)
    out = _sc_kernel(tab, idx, sp)
    return out.reshape(B, H, N, N)


# speedup vs baseline: 1.3409x; 1.0988x over previous
import functools

import jax
import jax.numpy as jnp
from jax import lax
from jax.experimental import pallas as pl
from jax.experimental.pallas import tpu as pltpu
from jax.experimental.pallas import tpu_sc as plsc

H = 32
V = 1537
VP = 1544
D = 5
F = 3
K = D * F
B, N = 16, 64
P = N * N
C = 64
HH = H // 2
TW = D * VP
PW = P // 2


def _fold_body(we_ref, w_ref, out_ref):
    out_ref[0] = jnp.dot(we_ref[...], w_ref[0], preferred_element_type=jnp.float32)


def _fold_tables(we_pad, w):
    return pl.pallas_call(
        _fold_body,
        grid=(D,),
        in_specs=[
            pl.BlockSpec((VP, H), lambda d: (0, 0)),
            pl.BlockSpec((1, H, H), lambda d: (d, 0, 0)),
        ],
        out_specs=pl.BlockSpec((1, VP, H), lambda d: (d, 0, 0)),
        out_shape=jax.ShapeDtypeStruct((D, VP, H), jnp.float32),
    )(we_pad, w)


_mesh = plsc.VectorSubcoreMesh(core_axis_name="c", subcore_axis_name="s")


@functools.partial(
    pl.kernel,
    out_type=jax.ShapeDtypeStruct((B, H, P), jnp.float32),
    mesh=_mesh,
    scratch_types=[
        pltpu.VMEM((TW * HH,), jnp.int32),
        pltpu.VMEM((2, C * K), jnp.int32),
        pltpu.VMEM((2, C), jnp.int32),
        pltpu.VMEM((2, H, C), jnp.float32),
        pltpu.SemaphoreType.DMA((2,)),
        pltpu.SemaphoreType.DMA((2,)),
        pltpu.SemaphoreType.DMA((2,)),
    ],
    compiler_params=pltpu.CompilerParams(
        needs_layout_passes=False, use_tc_tiling_on_sc=False),
)
def _sc_kernel(tab_hbm, idx_hbm, sp_hbm, out_hbm, tab_v, idx_v, sp_v, out_v,
               sem_i, sem_s, sem_o):
    cid = lax.axis_index("c")
    sid = lax.axis_index("s")
    wid = sid * 2 + cid
    b = wid % B
    ph = wid // B
    iota = lax.iota(jnp.int32, 16)
    iota_k = iota * K
    NCH = PW // C

    def in_copies(chunk, buf):
        base = ph * PW + chunk * C
        return (
            pltpu.make_async_copy(idx_hbm.at[b, pl.ds(base * K, C * K)],
                                  idx_v.at[buf], sem_i.at[buf]),
            pltpu.make_async_copy(sp_hbm.at[b, pl.ds(base, C)],
                                  sp_v.at[buf], sem_s.at[buf]),
        )

    def out_copy(chunk, buf):
        base = ph * PW + chunk * C
        return pltpu.make_async_copy(
            out_v.at[buf], out_hbm.at[b, pl.ds(0, H), pl.ds(base, C)],
            sem_o.at[buf])

    for c in (0, 1):
        for cp in in_copies(c, c):
            cp.start()
    pltpu.sync_copy(tab_hbm, tab_v)

    def chunk_pair(it, _):
        for par in (0, 1):
            chunk = it * 2 + par
            for cp in in_copies(chunk, par):
                cp.wait()

            @pl.when(it > 0)
            def _():
                out_copy(chunk - 2, par).wait()

            @plsc.parallel_loop(0, C // 16, unroll=2)
            def group_body(g):
                pos = g * 16 + iota
                vks = [
                    plsc.load_gather(idx_v.at[par], [g * (16 * K) + k + iota_k])
                    + (k // 3) * VP
                    for k in range(K)
                ]
                s = plsc.load_gather(sp_v.at[par], [pos])
                s = jnp.where(s == 0, 1, s)
                s = jnp.where(s > 1, s - 1, s)
                s = jnp.minimum(jnp.maximum(s, 0), D)
                scale = 1.0 / (3.0 * s.astype(jnp.float32))
                for hp in range(HH):
                    hrow = tab_v.at[pl.ds(hp * TW, TW)]
                    words = [plsc.bitcast(plsc.load_gather(hrow, [vks[k]]),
                                          jnp.bfloat16)
                             for k in range(K)]
                    paired = [words[i] + words[i + 1]
                              for i in range(0, K - 1, 2)]
                    paired.append(words[K - 1])
                    los, his = [], []
                    for wv in paired:
                        lo, hi = plsc.unpack(
                            wv, format=plsc.PackFormat.INTERLEAVED)
                        los.append(lo)
                        his.append(hi)
                    for vals in (los, his):
                        while len(vals) > 1:
                            vals[:] = [vals[i] + vals[i + 1]
                                       for i in range(0, len(vals) - 1, 2)] + (
                                [vals[-1]] if len(vals) % 2 else [])
                    out_v[par, hp, pl.ds(g * 16, 16)] = los[0] * scale
                    out_v[par, hp + HH, pl.ds(g * 16, 16)] = his[0] * scale

            out_copy(chunk, par).start()

            @pl.when(chunk + 2 < NCH)
            def _():
                for cp in in_copies(chunk + 2, par):
                    cp.start()

        return 0

    lax.fori_loop(0, NCH // 2, chunk_pair, 0)
    for par in (0, 1):
        out_copy(NCH - 2 + par, par).wait()


def kernel(attn_bias, spatial_pos, x, edge_input, attn_edge_type,
           edge_encoder_weight, edge_dis_encoder_weight):
    we_pad = jnp.pad(edge_encoder_weight, ((0, VP - V), (0, 0)))
    w = edge_dis_encoder_weight.reshape(-1, H, H)[:D]
    tc = _fold_tables(we_pad, w).reshape(TW, H)
    pairs = jnp.stack([tc[:, :HH], tc[:, HH:]], axis=-1).astype(jnp.bfloat16)
    tab = lax.bitcast_convert_type(
        pairs.transpose(1, 0, 2), jnp.int32).reshape(HH * TW)
    idx = edge_input.reshape(B, P * K).astype(jnp.int32)
    sp = spatial_pos.reshape(B, P).astype(jnp.int32)
    out = _sc_kernel(tab, idx, sp)
    return out.reshape(B, H, N, N)


# speedup vs baseline: 1.3587x; 1.0133x over previous
import functools

import jax
import jax.numpy as jnp
from jax import lax
from jax.experimental import pallas as pl
from jax.experimental.pallas import tpu as pltpu
from jax.experimental.pallas import tpu_sc as plsc

H = 32
V = 1537
VP = 1544
D = 5
F = 3
K = D * F
B, N = 16, 64
P = N * N
C = 64
HH = H // 2
TW = D * VP
PW = P // 2


def _fold_body(we_ref, w_ref, out_ref):
    out_ref[0] = jnp.dot(we_ref[...], w_ref[0], preferred_element_type=jnp.float32)


def _fold_tables(we_pad, w):
    return pl.pallas_call(
        _fold_body,
        grid=(D,),
        in_specs=[
            pl.BlockSpec((VP, H), lambda d: (0, 0)),
            pl.BlockSpec((1, H, H), lambda d: (d, 0, 0)),
        ],
        out_specs=pl.BlockSpec((1, VP, H), lambda d: (d, 0, 0)),
        out_shape=jax.ShapeDtypeStruct((D, VP, H), jnp.float32),
    )(we_pad, w)


_mesh = plsc.VectorSubcoreMesh(core_axis_name="c", subcore_axis_name="s")


@functools.partial(
    pl.kernel,
    out_type=jax.ShapeDtypeStruct((B, H, P), jnp.float32),
    mesh=_mesh,
    scratch_types=[
        pltpu.VMEM((TW * HH,), jnp.int32),
        pltpu.VMEM((2, C * K), jnp.int32),
        pltpu.VMEM((2, C), jnp.int32),
        pltpu.VMEM((2, H, C), jnp.float32),
        pltpu.SemaphoreType.DMA((2,)),
        pltpu.SemaphoreType.DMA((2,)),
        pltpu.SemaphoreType.DMA((2,)),
    ],
    compiler_params=pltpu.CompilerParams(
        needs_layout_passes=False, use_tc_tiling_on_sc=False),
)
def _sc_kernel(tab_hbm, idx_hbm, sp_hbm, out_hbm, tab_v, idx_v, sp_v, out_v,
               sem_i, sem_s, sem_o):
    cid = lax.axis_index("c")
    sid = lax.axis_index("s")
    wid = sid * 2 + cid
    b = wid % B
    ph = wid // B
    iota = lax.iota(jnp.int32, 16)
    iota_k = iota * K
    NCH = PW // C

    def in_copies(chunk, buf):
        base = ph * PW + chunk * C
        return (
            pltpu.make_async_copy(idx_hbm.at[b, pl.ds(base * K, C * K)],
                                  idx_v.at[buf], sem_i.at[buf]),
            pltpu.make_async_copy(sp_hbm.at[b, pl.ds(base, C)],
                                  sp_v.at[buf], sem_s.at[buf]),
        )

    def out_copy(chunk, buf):
        base = ph * PW + chunk * C
        return pltpu.make_async_copy(
            out_v.at[buf], out_hbm.at[b, pl.ds(0, H), pl.ds(base, C)],
            sem_o.at[buf])

    for c in (0, 1):
        for cp in in_copies(c, c):
            cp.start()
    pltpu.sync_copy(tab_hbm, tab_v)

    def chunk_pair(it, _):
        for par in (0, 1):
            chunk = it * 2 + par
            for cp in in_copies(chunk, par):
                cp.wait()

            @pl.when(it > 0)
            def _():
                out_copy(chunk - 2, par).wait()

            @plsc.parallel_loop(0, C // 16, unroll=2)
            def group_body(g):
                pos = g * 16 + iota
                vks = [
                    plsc.load_gather(idx_v.at[par], [g * (16 * K) + k + iota_k])
                    + (k // 3) * VP
                    for k in range(K)
                ]
                s = plsc.load_gather(sp_v.at[par], [pos])
                s = jnp.where(s == 0, 1, s)
                s = jnp.where(s > 1, s - 1, s)
                s = jnp.minimum(jnp.maximum(s, 0), D)
                scale = 1.0 / (3.0 * s.astype(jnp.float32))
                for hp in range(HH):
                    hrow = tab_v.at[pl.ds(hp * TW, TW)]
                    words = [plsc.bitcast(plsc.load_gather(hrow, [vks[k]]),
                                          jnp.bfloat16)
                             for k in range(K)]
                    paired = [words[i] + words[i + 1]
                              for i in range(0, K - 1, 2)]
                    paired.append(words[K - 1])
                    paired = [paired[i] + paired[i + 1]
                              for i in range(0, len(paired), 2)]
                    los, his = [], []
                    for wv in paired:
                        lo, hi = plsc.unpack(
                            wv, format=plsc.PackFormat.INTERLEAVED)
                        los.append(lo)
                        his.append(hi)
                    for vals in (los, his):
                        while len(vals) > 1:
                            vals[:] = [vals[i] + vals[i + 1]
                                       for i in range(0, len(vals) - 1, 2)] + (
                                [vals[-1]] if len(vals) % 2 else [])
                    out_v[par, hp, pl.ds(g * 16, 16)] = los[0] * scale
                    out_v[par, hp + HH, pl.ds(g * 16, 16)] = his[0] * scale

            out_copy(chunk, par).start()

            @pl.when(chunk + 2 < NCH)
            def _():
                for cp in in_copies(chunk + 2, par):
                    cp.start()

        return 0

    lax.fori_loop(0, NCH // 2, chunk_pair, 0)
    for par in (0, 1):
        out_copy(NCH - 2 + par, par).wait()


def kernel(attn_bias, spatial_pos, x, edge_input, attn_edge_type,
           edge_encoder_weight, edge_dis_encoder_weight):
    we_pad = jnp.pad(edge_encoder_weight, ((0, VP - V), (0, 0)))
    w = edge_dis_encoder_weight.reshape(-1, H, H)[:D]
    tc = _fold_tables(we_pad, w).reshape(TW, H)
    pairs = jnp.stack([tc[:, :HH], tc[:, HH:]], axis=-1).astype(jnp.bfloat16)
    tab = lax.bitcast_convert_type(
        pairs.transpose(1, 0, 2), jnp.int32).reshape(HH * TW)
    idx = edge_input.reshape(B, P * K).astype(jnp.int32)
    sp = spatial_pos.reshape(B, P).astype(jnp.int32)
    out = _sc_kernel(tab, idx, sp)
    return out.reshape(B, H, N, N)


# speedup vs baseline: 2.1801x; 1.6046x over previous
import functools

import jax
import jax.numpy as jnp
from jax import lax
from jax.experimental import pallas as pl
from jax.experimental.pallas import tpu as pltpu
from jax.experimental.pallas import tpu_sc as plsc

H = 32
V = 1537
VP = 1544
D = 5
F = 3
K = D * F
KW = 8
B, N = 16, 64
P = N * N
C = 128
HH = H // 2
TW = D * V
TWP = TW + 3
PW = P // 2
NCH = PW // C


def _fold_body(we_ref, w_ref, out_ref):
    out_ref[0] = jnp.dot(we_ref[...], w_ref[0], preferred_element_type=jnp.float32)


def _fold_tables(we_pad, w):
    return pl.pallas_call(
        _fold_body,
        grid=(D,),
        in_specs=[
            pl.BlockSpec((VP, H), lambda d: (0, 0)),
            pl.BlockSpec((1, H, H), lambda d: (d, 0, 0)),
        ],
        out_specs=pl.BlockSpec((1, VP, H), lambda d: (d, 0, 0)),
        out_shape=jax.ShapeDtypeStruct((D, VP, H), jnp.float32),
    )(we_pad, w)


_mesh = plsc.VectorSubcoreMesh(core_axis_name="c", subcore_axis_name="s")


@functools.partial(
    pl.kernel,
    out_type=jax.ShapeDtypeStruct((B, HH, P), jnp.int32),
    mesh=_mesh,
    scratch_types=[
        pltpu.VMEM((TWP * HH,), jnp.int32),
        pltpu.VMEM((2, KW, C), jnp.int32),
        pltpu.VMEM((2, HH, C), jnp.int32),
        pltpu.SemaphoreType.DMA((2,)),
        pltpu.SemaphoreType.DMA((2,)),
    ],
    compiler_params=pltpu.CompilerParams(needs_layout_passes=False),
)
def _sc_kernel(tab_hbm, ids_hbm, out_hbm, tab_v, ids_v, out_v, sem_i, sem_o):
    cid = lax.axis_index("c")
    sid = lax.axis_index("s")
    wid = sid * 2 + cid
    b = wid % B
    ph = wid // B

    def in_copy(chunk, buf):
        base = ph * PW + chunk * C
        return pltpu.make_async_copy(
            ids_hbm.at[b, pl.ds(0, KW), pl.ds(base, C)],
            ids_v.at[buf], sem_i.at[buf])

    def out_copy(chunk, buf):
        base = ph * PW + chunk * C
        return pltpu.make_async_copy(
            out_v.at[buf], out_hbm.at[b, pl.ds(0, HH), pl.ds(base, C)],
            sem_o.at[buf])

    in_copy(0, 0).start()
    in_copy(1, 1).start()
    pltpu.sync_copy(tab_hbm, tab_v)

    def chunk_pair(it, _):
        for par in (0, 1):
            chunk = it * 2 + par
            in_copy(chunk, par).wait()

            @pl.when(it > 0)
            def _():
                out_copy(chunk - 2, par).wait()

            @plsc.parallel_loop(0, C // 16, unroll=2)
            def group_body(g):
                words = [ids_v[par, j, pl.ds(g * 16, 16)] for j in range(KW)]
                vks = []
                for k in range(K):
                    raw = (words[k // 2] & 0xFFFF if k % 2 == 0
                           else lax.shift_right_logical(words[k // 2], 16))
                    vks.append(raw + (k // 3) * V)
                s = lax.shift_right_logical(words[KW - 1], 16)
                s = jnp.where(s == 0, 1, s)
                s = jnp.where(s > 1, s - 1, s)
                s = jnp.minimum(jnp.maximum(s, 0), D)
                scale = 1.0 / (3.0 * s.astype(jnp.float32))
                for hp in range(HH):
                    hrow = tab_v.at[pl.ds(hp * TWP, TW)]
                    wvs = [plsc.bitcast(plsc.load_gather(hrow, [vks[k]]),
                                        jnp.bfloat16)
                           for k in range(K)]
                    paired = [wvs[i] + wvs[i + 1] for i in range(0, K - 1, 2)]
                    paired.append(wvs[K - 1])
                    paired = [paired[i] + paired[i + 1]
                              for i in range(0, len(paired), 2)]
                    los, his = [], []
                    for wv in paired:
                        lo, hi = plsc.unpack(
                            wv, format=plsc.PackFormat.INTERLEAVED)
                        los.append(lo)
                        his.append(hi)
                    for vals in (los, his):
                        while len(vals) > 1:
                            vals[:] = [vals[i] + vals[i + 1]
                                       for i in range(0, len(vals) - 1, 2)] + (
                                [vals[-1]] if len(vals) % 2 else [])
                    packed = plsc.bitcast(
                        plsc.pack(los[0] * scale, his[0] * scale,
                                  format=plsc.PackFormat.INTERLEAVED),
                        jnp.int32)
                    out_v[par, hp, pl.ds(g * 16, 16)] = packed

            out_copy(chunk, par).start()

            @pl.when(chunk + 2 < NCH)
            def _():
                in_copy(chunk + 2, par).start()

        return 0

    lax.fori_loop(0, NCH // 2, chunk_pair, 0)
    for par in (0, 1):
        out_copy(NCH - 2 + par, par).wait()


def kernel(attn_bias, spatial_pos, x, edge_input, attn_edge_type,
           edge_encoder_weight, edge_dis_encoder_weight):
    we_pad = jnp.pad(edge_encoder_weight, ((0, VP - V), (0, 0)))
    w = edge_dis_encoder_weight.reshape(-1, H, H)[:D]
    tc = _fold_tables(we_pad, w)[:, :V, :].reshape(TW, H)
    pairs = jnp.stack([tc[:, :HH], tc[:, HH:]], axis=-1).astype(jnp.bfloat16)
    tab = lax.bitcast_convert_type(
        pairs.transpose(1, 0, 2), jnp.int32)
    tab = jnp.pad(tab, ((0, 0), (0, TWP - TW))).reshape(HH * TWP)
    ids16 = jnp.concatenate(
        [edge_input.reshape(B, P, K).astype(jnp.int16),
         spatial_pos.reshape(B, P, 1).astype(jnp.int16)], axis=-1)
    ids = lax.bitcast_convert_type(
        ids16.reshape(B, P, KW, 2), jnp.int32).transpose(0, 2, 1)
    out = _sc_kernel(tab, ids)
    vals = lax.bitcast_convert_type(out, jnp.bfloat16).astype(jnp.float32)
    out = vals.transpose(0, 3, 1, 2).reshape(B, H, P)
    return out.reshape(B, H, N, N)
